# baseline (speedup 1.0000x reference)
import functools

import jax
import jax.numpy as jnp
from jax import lax
from jax.experimental import pallas as pl
from jax.experimental.pallas import tpu as pltpu
from jax.experimental.pallas import tpu_sc as plsc

_D = 128
_MAX_ID = 1025
_ROWS_PAD = 1032
_NC, _NS, _LANES = 2, 16, 16
_NW = _NC * _NS
_B, _L = 4096, 200
_TOT = _B * _L
_CHUNK = 128
_PER_W = _TOT // _NW
_NCH = _PER_W // _CHUNK


def _project_body(t_ref, w_ref, b_ref, m_ref, o_ref):
    prod = lax.dot_general(
        t_ref[...], w_ref[...], (((1,), (1,)), ((), ())),
        preferred_element_type=jnp.float32)
    o_ref[...] = m_ref[...] * (prod + b_ref[...])


def _project_table(table_padded, w, b, mix):
    return pl.pallas_call(
        _project_body,
        out_shape=jax.ShapeDtypeStruct((_ROWS_PAD, _D), jnp.float32),
    )(table_padded, w, b.reshape(1, _D), mix.reshape(1, 1))


_MESH = plsc.VectorSubcoreMesh(
    core_axis_name="c", subcore_axis_name="s",
    num_cores=_NC, num_subcores=_NS)


@functools.partial(
    pl.kernel,
    out_type=jax.ShapeDtypeStruct((_NW, _NCH, _CHUNK, _D), jnp.float32),
    mesh=_MESH,
    scratch_types=[
        pltpu.VMEM((_NCH, _CHUNK), jnp.int32),
        pltpu.VMEM((4, _CHUNK, _D), jnp.float32),
        pltpu.VMEM_SHARED((_ROWS_PAD, _D), jnp.float32),
        pltpu.SemaphoreType.DMA,
        pltpu.SemaphoreType.DMA,
        pltpu.SemaphoreType.DMA,
        pltpu.SemaphoreType.DMA,
        pltpu.SemaphoreType.DMA,
        pltpu.SemaphoreType.DMA,
        pltpu.SemaphoreType.DMA,
        pltpu.SemaphoreType.DMA,
    ],
)
def _gather_rows(table_hbm, idx_hbm, out_hbm, idx_v, rows_v, tsh,
                 g0, g1, g2, g3, o0, o1, o2, o3):
    sid = lax.axis_index("s")
    wid = sid * _NC + lax.axis_index("c")

    @pl.when(sid == 0)
    def _stage():
        pltpu.sync_copy(table_hbm, tsh)

    pltpu.sync_copy(idx_hbm.at[wid], idx_v)

    _VPC = _CHUNK // _LANES

    def _clamp(i, carry):
        c = i // _VPC
        j = (i % _VPC) * _LANES
        v = idx_v[c, pl.ds(j, _LANES)]
        idx_v[c, pl.ds(j, _LANES)] = jnp.minimum(
            jnp.maximum(v, 0), _MAX_ID)
        return carry

    lax.fori_loop(0, _NCH * _VPC, _clamp, 0)
    plsc.subcore_barrier()

    def _g_start(c, buf, sem):
        @pl.when(c % 8 == 7)
        def _():
            pltpu.make_async_copy(
                table_hbm.at[idx_v.at[c]], rows_v.at[buf], sem).start()

        @pl.when(c % 8 != 7)
        def _():
            pltpu.make_async_copy(
                tsh.at[idx_v.at[c]], rows_v.at[buf], sem).start()

    def _g(c, buf, sem):
        return pltpu.make_async_copy(tsh.at[idx_v.at[c]], rows_v.at[buf], sem)

    def _w(c, buf, sem):
        return pltpu.make_async_copy(rows_v.at[buf], out_hbm.at[wid, c], sem)

    _g_start(0, 0, g0)
    _g_start(1, 1, g1)

    _NQ = _NCH // 4

    def _quad(q, carry):
        c = 4 * q

        _g(c, 0, g0).wait()
        _w(c, 0, o0).start()

        @pl.when(q > 0)
        def _():
            _w(c - 2, 2, o2).wait()
        _g_start(c + 2, 2, g2)

        _g(c + 1, 1, g1).wait()
        _w(c + 1, 1, o1).start()

        @pl.when(q > 0)
        def _():
            _w(c - 1, 3, o3).wait()
        _g_start(c + 3, 3, g3)

        _g(c + 2, 2, g2).wait()
        _w(c + 2, 2, o2).start()

        @pl.when(q < _NQ - 1)
        def _():
            _w(c, 0, o0).wait()
            _g_start(c + 4, 0, g0)

        _g(c + 3, 3, g3).wait()
        _w(c + 3, 3, o3).start()

        @pl.when(q < _NQ - 1)
        def _():
            _w(c + 1, 1, o1).wait()
            _g_start(c + 5, 1, g1)

        return carry

    lax.fori_loop(0, _NQ, _quad, 0)
    _w(_NCH - 4, 0, o0).wait()
    _w(_NCH - 3, 1, o1).wait()
    _w(_NCH - 2, 2, o2).wait()
    _w(_NCH - 1, 3, o3).wait()


def kernel(counter_values, c_emb_weight, read_proj_w, read_proj_b, mix):
    table_padded = jnp.pad(
        c_emb_weight, ((0, _ROWS_PAD - c_emb_weight.shape[0]), (0, 0)))
    ptab = _project_table(table_padded, read_proj_w, read_proj_b,
                          jnp.asarray(mix, jnp.float32))
    idx = counter_values.reshape(_NW, _NCH, _CHUNK)
    out = _gather_rows(ptab, idx)
    return out.reshape(_B, _L, _D)


# speedup vs baseline: 1.1009x; 1.1009x over previous
import functools

import jax
import jax.numpy as jnp
from jax import lax
from jax.experimental import pallas as pl
from jax.experimental.pallas import tpu as pltpu
from jax.experimental.pallas import tpu_sc as plsc

_D = 128
_MAX_ID = 1025
_ROWS_PAD = 1032
_NC, _NS, _LANES = 2, 16, 16
_NW = _NC * _NS
_B, _L = 4096, 200
_TOT = _B * _L
_CHUNK = 128
_PER_W = _TOT // _NW
_NCH = _PER_W // _CHUNK


def _project_body(t_ref, w_ref, b_ref, m_ref, o_ref):
    prod = lax.dot_general(
        t_ref[...], w_ref[...], (((1,), (1,)), ((), ())),
        preferred_element_type=jnp.float32)
    o_ref[...] = m_ref[...] * (prod + b_ref[...])


def _project_table(table_padded, w, b, mix):
    return pl.pallas_call(
        _project_body,
        out_shape=jax.ShapeDtypeStruct((_ROWS_PAD, _D), jnp.float32),
    )(table_padded, w, b.reshape(1, _D), mix.reshape(1, 1))


_MESH = plsc.VectorSubcoreMesh(
    core_axis_name="c", subcore_axis_name="s",
    num_cores=_NC, num_subcores=_NS)


@functools.partial(
    pl.kernel,
    out_type=jax.ShapeDtypeStruct((_NW, _NCH, _CHUNK, _D), jnp.float32),
    mesh=_MESH,
    scratch_types=[
        pltpu.VMEM((_NCH, _CHUNK), jnp.int32),
        pltpu.VMEM((4, _CHUNK, _D), jnp.float32),
        pltpu.VMEM_SHARED((_ROWS_PAD, _D), jnp.float32),
        pltpu.SemaphoreType.DMA,
        pltpu.SemaphoreType.DMA,
        pltpu.SemaphoreType.DMA,
        pltpu.SemaphoreType.DMA,
        pltpu.SemaphoreType.DMA,
        pltpu.SemaphoreType.DMA,
        pltpu.SemaphoreType.DMA,
        pltpu.SemaphoreType.DMA,
    ],
)
def _gather_rows(table_hbm, idx_hbm, out_hbm, idx_v, rows_v, tsh,
                 g0, g1, g2, g3, o0, o1, o2, o3):
    sid = lax.axis_index("s")
    wid = sid * _NC + lax.axis_index("c")

    @pl.when(sid == 0)
    def _stage():
        pltpu.sync_copy(table_hbm, tsh)

    pltpu.sync_copy(idx_hbm.at[wid], idx_v)

    _VPC = _CHUNK // _LANES

    def _clamp(i, carry):
        c = i // _VPC
        j = (i % _VPC) * _LANES
        v = idx_v[c, pl.ds(j, _LANES)]
        idx_v[c, pl.ds(j, _LANES)] = jnp.minimum(
            jnp.maximum(v, 0), _MAX_ID)
        return carry

    lax.fori_loop(0, _NCH * _VPC, _clamp, 0)
    plsc.subcore_barrier()

    def _g_start(c, buf, sem):
        @pl.when(c % 24 == 23)
        def _():
            pltpu.make_async_copy(
                table_hbm.at[idx_v.at[c]], rows_v.at[buf], sem).start()

        @pl.when(c % 24 != 23)
        def _():
            pltpu.make_async_copy(
                tsh.at[idx_v.at[c]], rows_v.at[buf], sem).start()

    def _g(c, buf, sem):
        return pltpu.make_async_copy(tsh.at[idx_v.at[c]], rows_v.at[buf], sem)

    def _w(c, buf, sem):
        return pltpu.make_async_copy(rows_v.at[buf], out_hbm.at[wid, c], sem)

    _g_start(0, 0, g0)
    _g_start(1, 1, g1)

    _NQ = _NCH // 4

    def _quad(q, carry):
        c = 4 * q

        _g(c, 0, g0).wait()
        _w(c, 0, o0).start()

        @pl.when(q > 0)
        def _():
            _w(c - 2, 2, o2).wait()
        _g_start(c + 2, 2, g2)

        _g(c + 1, 1, g1).wait()
        _w(c + 1, 1, o1).start()

        @pl.when(q > 0)
        def _():
            _w(c - 1, 3, o3).wait()
        _g_start(c + 3, 3, g3)

        _g(c + 2, 2, g2).wait()
        _w(c + 2, 2, o2).start()

        @pl.when(q < _NQ - 1)
        def _():
            _w(c, 0, o0).wait()
            _g_start(c + 4, 0, g0)

        _g(c + 3, 3, g3).wait()
        _w(c + 3, 3, o3).start()

        @pl.when(q < _NQ - 1)
        def _():
            _w(c + 1, 1, o1).wait()
            _g_start(c + 5, 1, g1)

        return carry

    lax.fori_loop(0, _NQ, _quad, 0)
    _w(_NCH - 4, 0, o0).wait()
    _w(_NCH - 3, 1, o1).wait()
    _w(_NCH - 2, 2, o2).wait()
    _w(_NCH - 1, 3, o3).wait()


def kernel(counter_values, c_emb_weight, read_proj_w, read_proj_b, mix):
    table_padded = jnp.pad(
        c_emb_weight, ((0, _ROWS_PAD - c_emb_weight.shape[0]), (0, 0)))
    ptab = _project_table(table_padded, read_proj_w, read_proj_b,
                          jnp.asarray(mix, jnp.float32))
    idx = counter_values.reshape(_NW, _NCH, _CHUNK)
    out = _gather_rows(ptab, idx)
    return out.reshape(_B, _L, _D)


# speedup vs baseline: 1.1323x; 1.0285x over previous
import functools

import jax
import jax.numpy as jnp
from jax import lax
from jax.experimental import pallas as pl
from jax.experimental.pallas import tpu as pltpu
from jax.experimental.pallas import tpu_sc as plsc

_D = 128
_MAX_ID = 1025
_ROWS_PAD = 1032
_NC, _NS, _LANES = 2, 16, 16
_NW = _NC * _NS
_B, _L = 4096, 200
_TOT = _B * _L
_CHUNK = 128
_PER_W = _TOT // _NW
_NCH = _PER_W // _CHUNK


def _project_body(t_ref, w_ref, b_ref, m_ref, o_ref):
    prod = lax.dot_general(
        t_ref[...], w_ref[...], (((1,), (1,)), ((), ())),
        preferred_element_type=jnp.float32)
    o_ref[...] = m_ref[...] * (prod + b_ref[...])


def _project_table(table_padded, w, b, mix):
    return pl.pallas_call(
        _project_body,
        out_shape=jax.ShapeDtypeStruct((_ROWS_PAD, _D), jnp.float32),
    )(table_padded, w, b.reshape(1, _D), mix.reshape(1, 1))


_MESH = plsc.VectorSubcoreMesh(
    core_axis_name="c", subcore_axis_name="s",
    num_cores=_NC, num_subcores=_NS)


@functools.partial(
    pl.kernel,
    out_type=jax.ShapeDtypeStruct((_NW, _NCH, _CHUNK, _D), jnp.float32),
    mesh=_MESH,
    scratch_types=[
        pltpu.VMEM((_NCH, _CHUNK), jnp.int32),
        pltpu.VMEM((4, _CHUNK, _D), jnp.float32),
        pltpu.VMEM_SHARED((_ROWS_PAD, _D), jnp.float32),
        pltpu.SemaphoreType.DMA,
        pltpu.SemaphoreType.DMA,
        pltpu.SemaphoreType.DMA,
        pltpu.SemaphoreType.DMA,
        pltpu.SemaphoreType.DMA,
        pltpu.SemaphoreType.DMA,
        pltpu.SemaphoreType.DMA,
        pltpu.SemaphoreType.DMA,
    ],
)
def _gather_rows(table_hbm, idx_hbm, out_hbm, idx_v, rows_v, tsh,
                 g0, g1, g2, g3, o0, o1, o2, o3):
    sid = lax.axis_index("s")
    wid = sid * _NC + lax.axis_index("c")

    @pl.when(sid == 0)
    def _stage():
        pltpu.sync_copy(table_hbm, tsh)

    pltpu.sync_copy(idx_hbm.at[wid], idx_v)

    _VPC = _CHUNK // _LANES

    def _clamp(i, carry):
        c = i // _VPC
        j = (i % _VPC) * _LANES
        v = idx_v[c, pl.ds(j, _LANES)]
        idx_v[c, pl.ds(j, _LANES)] = jnp.minimum(
            jnp.maximum(v, 0), _MAX_ID)
        return carry

    lax.fori_loop(0, 6 * _VPC, _clamp, 0)

    def _clamp_chunk(t):
        @pl.when(t < _NCH)
        def _():
            def _cl(j, carry):
                v = idx_v[t, pl.ds(j * _LANES, _LANES)]
                idx_v[t, pl.ds(j * _LANES, _LANES)] = jnp.minimum(
                    jnp.maximum(v, 0), _MAX_ID)
                return carry

            lax.fori_loop(0, _VPC, _cl, 0)

    plsc.subcore_barrier()

    def _g_start(c, buf, sem):
        @pl.when(c % 16 == 15)
        def _():
            pltpu.make_async_copy(
                table_hbm.at[idx_v.at[c]], rows_v.at[buf], sem).start()

        @pl.when(c % 16 != 15)
        def _():
            pltpu.make_async_copy(
                tsh.at[idx_v.at[c]], rows_v.at[buf], sem).start()

    def _g(c, buf, sem):
        return pltpu.make_async_copy(tsh.at[idx_v.at[c]], rows_v.at[buf], sem)

    def _w(c, buf, sem):
        return pltpu.make_async_copy(rows_v.at[buf], out_hbm.at[wid, c], sem)

    _g_start(0, 0, g0)
    _g_start(1, 1, g1)

    _NQ = _NCH // 4

    def _quad(q, carry):
        c = 4 * q

        _clamp_chunk(c + 6)
        _clamp_chunk(c + 7)
        _clamp_chunk(c + 8)
        _clamp_chunk(c + 9)

        _g(c, 0, g0).wait()
        _w(c, 0, o0).start()

        @pl.when(q > 0)
        def _():
            _w(c - 2, 2, o2).wait()
        _g_start(c + 2, 2, g2)

        _g(c + 1, 1, g1).wait()
        _w(c + 1, 1, o1).start()

        @pl.when(q > 0)
        def _():
            _w(c - 1, 3, o3).wait()
        _g_start(c + 3, 3, g3)

        _g(c + 2, 2, g2).wait()
        _w(c + 2, 2, o2).start()

        @pl.when(q < _NQ - 1)
        def _():
            _w(c, 0, o0).wait()
            _g_start(c + 4, 0, g0)

        _g(c + 3, 3, g3).wait()
        _w(c + 3, 3, o3).start()

        @pl.when(q < _NQ - 1)
        def _():
            _w(c + 1, 1, o1).wait()
            _g_start(c + 5, 1, g1)

        return carry

    lax.fori_loop(0, _NQ, _quad, 0)
    _w(_NCH - 4, 0, o0).wait()
    _w(_NCH - 3, 1, o1).wait()
    _w(_NCH - 2, 2, o2).wait()
    _w(_NCH - 1, 3, o3).wait()


def kernel(counter_values, c_emb_weight, read_proj_w, read_proj_b, mix):
    table_padded = jnp.pad(
        c_emb_weight, ((0, _ROWS_PAD - c_emb_weight.shape[0]), (0, 0)))
    ptab = _project_table(table_padded, read_proj_w, read_proj_b,
                          jnp.asarray(mix, jnp.float32))
    idx = counter_values.reshape(_NW, _NCH, _CHUNK)
    out = _gather_rows(ptab, idx)
    return out.reshape(_B, _L, _D)
